# baseline (speedup 1.0000x reference)
import jax
import jax.numpy as jnp
from jax.experimental import pallas as pl
from jax.experimental.pallas import tpu as pltpu

_EPS = 0.01
_N = 4096
_BLK = 512


def _toz_body(x_ref, o_ref):
    i = pl.program_id(0)
    o_ref[...] = jnp.zeros((_BLK, 1, 64, 64), jnp.float32)

    G = _BLK // 64
    @pl.when(i < _N // _BLK)
    def _():
        slab = jnp.where(
            jax.lax.broadcasted_iota(jnp.int32, (64, 1, 1, 64), 0)
            == jax.lax.broadcasted_iota(jnp.int32, (64, 1, 1, 64), 3),
            _EPS, 0.0).astype(jnp.float32)
        for g in range(G - 1):
            o_ref[pl.ds(64 * g + 1, 64), :, pl.ds(G * i + g, 1), :] = slab
        o_ref[pl.ds(64 * (G - 1) + 1, 63), :, pl.ds(G * i + G - 1, 1), :] = (
            slab[:63])

    @pl.when(i > 0)
    def _():
        o_ref[pl.ds(0, 1), :, pl.ds(G * i - 1, 1), :] = jnp.where(
            jax.lax.broadcasted_iota(jnp.int32, (1, 1, 1, 64), 3) == 63,
            _EPS, 0.0).astype(jnp.float32)

    @pl.when(i == 0)
    def _():
        o_ref[pl.ds(0, 1), :, :, :] = x_ref[...]


def kernel(x):
    grid = (_N + 1 + _BLK - 1) // _BLK
    out = pl.pallas_call(
        _toz_body,
        grid=(grid,),
        in_specs=[pl.BlockSpec((1, 1, 64, 64), lambda i: (0, 0, 0, 0))],
        out_specs=pl.BlockSpec((_BLK, 1, 64, 64), lambda i: (i, 0, 0, 0)),
        out_shape=jax.ShapeDtypeStruct((_N + 1, 1, 64, 64), jnp.float32),
    )(x)
    return out


# speedup vs baseline: 1.1074x; 1.1074x over previous
import jax
import jax.numpy as jnp
from jax import lax
from jax.experimental import pallas as pl
from jax.experimental.pallas import tpu as pltpu
from jax.experimental.pallas import tpu_sc as plsc

_EPS = 0.01
_N = 4096
_NW = 32
_ROWS_PER_W = _N // _NW
_BLK = 4
_NBLK = _ROWS_PER_W // _BLK

_mesh = plsc.VectorSubcoreMesh(core_axis_name="c", subcore_axis_name="s")


def _toz_sc_body(x_hbm, zeros_hbm, out_hbm, buf0, buf1, xbuf, sem0, sem1):
    wid = lax.axis_index("s") * 2 + lax.axis_index("c")
    lane = lax.iota(jnp.int32, 16)
    mask = lane < _BLK
    eps_v = jnp.full((16,), _EPS, jnp.float32)
    zero_v = jnp.zeros((16,), jnp.float32)

    @pl.when(wid == 0)
    def _():
        pltpu.sync_copy(x_hbm, xbuf)
        pltpu.sync_copy(xbuf, out_hbm.at[pl.ds(0, 1)])

    pltpu.sync_copy(zeros_hbm, buf0)
    pltpu.sync_copy(zeros_hbm, buf1)

    bufs = (buf0, buf1)
    sems = (sem0, sem1)
    copies = [None, None]
    prev_idx = [None, None]

    for t in range(_NBLK):
        b = t % 2
        buf = bufs[b]
        if copies[b] is not None:
            copies[b].wait()
            plsc.store_scatter(buf, prev_idx[b], zero_v, mask=mask)
        j0 = 1 + wid * _ROWS_PER_W + t * _BLK
        fcode = j0 - 1 + lane
        idx = [lane,
               lax.shift_right_logical(fcode, 6),
               lax.bitwise_and(fcode, 63)]
        plsc.store_scatter(buf, idx, eps_v, mask=mask)
        copies[b] = pltpu.async_copy(buf, out_hbm.at[pl.ds(j0, _BLK)], sems[b])
        prev_idx[b] = idx

    copies[0].wait()
    copies[1].wait()


def kernel(x):
    xf = x.reshape(1, 64, 64)
    zeros = jnp.zeros((_BLK, 64, 64), jnp.float32)
    out = pl.kernel(
        _toz_sc_body,
        out_type=jax.ShapeDtypeStruct((_N + 1, 64, 64), jnp.float32),
        mesh=_mesh,
        compiler_params=pltpu.CompilerParams(needs_layout_passes=False),
        scratch_types=[
            pltpu.VMEM((_BLK, 64, 64), jnp.float32),
            pltpu.VMEM((_BLK, 64, 64), jnp.float32),
            pltpu.VMEM((1, 64, 64), jnp.float32),
            pltpu.SemaphoreType.DMA,
            pltpu.SemaphoreType.DMA,
        ],
    )(xf, zeros)
    return out.reshape(_N + 1, 1, 64, 64)


# speedup vs baseline: 1.2064x; 1.0894x over previous
import jax
import jax.numpy as jnp
from jax import lax
from jax.experimental import pallas as pl
from jax.experimental.pallas import tpu as pltpu

_EPS = 0.01
_N = 4096
_R = (_N + 1) * 64
_JBLK = 128
_BLK = _JBLK * 64


def _toz_body(x_ref, o_ref):
    i = pl.program_id(0)
    o_ref[...] = jnp.zeros((_BLK, 64), jnp.float32)

    lane = lax.broadcasted_iota(jnp.int32, (1, 64), 1)

    def poke(j, _):
        fcode = j - 1
        r = lax.shift_right_logical(fcode, 6)
        c = lax.bitwise_and(fcode, 63)
        gl = 64 * j + r - i * _BLK
        o_ref[pl.ds(gl, 1), :] = jnp.where(lane == c, _EPS, 0.0).astype(
            jnp.float32)
        return 0

    j_lo = jnp.maximum(1, i * _JBLK)
    j_hi = jnp.minimum(_N + 1, (i + 1) * _JBLK)
    lax.fori_loop(j_lo, j_hi, poke, 0)

    @pl.when(i == 0)
    def _():
        o_ref[pl.ds(0, 64), :] = x_ref[...]


def kernel(x):
    xf = x.reshape(64, 64)
    grid = (_R + _BLK - 1) // _BLK
    out = pl.pallas_call(
        _toz_body,
        grid=(grid,),
        in_specs=[pl.BlockSpec((64, 64), lambda i: (0, 0))],
        out_specs=pl.BlockSpec((_BLK, 64), lambda i: (i, 0)),
        out_shape=jax.ShapeDtypeStruct((_R, 64), jnp.float32),
    )(xf)
    return out.reshape(_N + 1, 1, 64, 64)
